# baseline (speedup 1.0000x reference)
import functools

import jax
import jax.numpy as jnp
from jax.experimental import pallas as pl
from jax.experimental.pallas import tpu as pltpu

_BB = 4


def _body(x_ref, phi_ref, ehi_ref, ctab_ref, out_ref, scr, sem):
    step = pl.program_id(0)
    nsteps = pl.num_programs(0)
    zrow = jnp.zeros((1, x_ref.shape[2]), dtype=jnp.bfloat16)
    for i in range(_BB):
        slot = i % 2
        g = step * _BB + i

        @pl.when(jnp.logical_or(step > 0, i >= 2))
        def _wait():
            pltpu.make_async_copy(
                scr.at[slot], out_ref.at[g - 2], sem.at[slot]).wait()

        vp = x_ref[i, 0:1, :]
        ve = x_ref[i, 1:2, :]
        g_p = (phi_ref[:, :] >= vp).astype(jnp.bfloat16)
        g_e = (ehi_ref[:, :] >= ve).astype(jnp.bfloat16)
        oh_p = g_p - jnp.concatenate([zrow, g_p[:-1, :]], axis=0)
        oh_e = g_e - jnp.concatenate([zrow, g_e[:-1, :]], axis=0)
        oh = jnp.concatenate([oh_p, oh_e], axis=0)
        scr[slot] = jnp.dot(ctab_ref[:, :], oh,
                            preferred_element_type=jnp.float32)
        pltpu.make_async_copy(scr.at[slot], out_ref.at[g], sem.at[slot]).start()

    @pl.when(step == nsteps - 1)
    def _drain():
        last = nsteps * _BB
        for slot in range(2):
            pltpu.make_async_copy(
                scr.at[slot], out_ref.at[last - 2 + slot], sem.at[slot]).wait()


@functools.partial(jax.jit, static_argnames=("interpret",))
def kernel(x, pitch_bins, energy_bins, pitch_embedding, energy_embedding,
           interpret=False):
    B, _, T = x.shape
    N, H = pitch_embedding.shape
    Tt = T

    inf = jnp.array([jnp.inf], dtype=jnp.float32)
    p_hi = jnp.concatenate([pitch_bins, inf])[:, None]
    e_hi = jnp.concatenate([energy_bins, inf])[:, None]
    ctab = jnp.concatenate(
        [pitch_embedding.T, energy_embedding.T], axis=1,
    ).astype(jnp.bfloat16)

    grid = (B // _BB,)
    return pl.pallas_call(
        _body,
        grid=grid,
        in_specs=[
            pl.BlockSpec((_BB, 2, Tt), lambda b: (b, 0, 0)),
            pl.BlockSpec((N, 1), lambda b: (0, 0)),
            pl.BlockSpec((N, 1), lambda b: (0, 0)),
            pl.BlockSpec((H, 2 * N), lambda b: (0, 0)),
        ],
        out_specs=pl.BlockSpec(memory_space=pl.MemorySpace.ANY),
        out_shape=jax.ShapeDtypeStruct((B, H, T), jnp.float32),
        scratch_shapes=[
            pltpu.VMEM((2, H, Tt), jnp.float32),
            pltpu.SemaphoreType.DMA((2,)),
        ],
        compiler_params=pltpu.CompilerParams(
            dimension_semantics=("arbitrary",)),
        interpret=interpret,
    )(x, p_hi, e_hi, ctab)


# speedup vs baseline: 1.0254x; 1.0254x over previous
import functools

import jax
import jax.numpy as jnp
from jax.experimental import pallas as pl
from jax.experimental.pallas import tpu as pltpu

_BB = 4


def _body(x_ref, phi_ref, ehi_ref, ctab_ref, out_ref, scr, sem):
    step = pl.program_id(0)
    nsteps = pl.num_programs(0)
    zrow = jnp.zeros((1, x_ref.shape[2]), dtype=jnp.bfloat16)
    for i in range(_BB):
        slot = i % 4
        g = step * _BB + i

        @pl.when(step > 0)
        def _wait():
            pltpu.make_async_copy(
                scr.at[slot], out_ref.at[g - 4], sem.at[slot]).wait()

        vp = x_ref[i, 0:1, :]
        ve = x_ref[i, 1:2, :]
        g_p = (phi_ref[:, :] >= vp).astype(jnp.bfloat16)
        g_e = (ehi_ref[:, :] >= ve).astype(jnp.bfloat16)
        oh_p = g_p - jnp.concatenate([zrow, g_p[:-1, :]], axis=0)
        oh_e = g_e - jnp.concatenate([zrow, g_e[:-1, :]], axis=0)
        oh = jnp.concatenate([oh_p, oh_e], axis=0)
        scr[slot] = jnp.dot(ctab_ref[:, :], oh,
                            preferred_element_type=jnp.float32)
        pltpu.make_async_copy(scr.at[slot], out_ref.at[g], sem.at[slot]).start()

    @pl.when(step == nsteps - 1)
    def _drain():
        last = nsteps * _BB
        for slot in range(4):
            pltpu.make_async_copy(
                scr.at[slot], out_ref.at[last - 4 + slot], sem.at[slot]).wait()


@functools.partial(jax.jit, static_argnames=("interpret",))
def kernel(x, pitch_bins, energy_bins, pitch_embedding, energy_embedding,
           interpret=False):
    B, _, T = x.shape
    N, H = pitch_embedding.shape
    Tt = T

    inf = jnp.array([jnp.inf], dtype=jnp.float32)
    p_hi = jnp.concatenate([pitch_bins, inf])[:, None]
    e_hi = jnp.concatenate([energy_bins, inf])[:, None]
    ctab = jnp.concatenate(
        [pitch_embedding.T, energy_embedding.T], axis=1,
    ).astype(jnp.bfloat16)

    grid = (B // _BB,)
    return pl.pallas_call(
        _body,
        grid=grid,
        in_specs=[
            pl.BlockSpec((_BB, 2, Tt), lambda b: (b, 0, 0)),
            pl.BlockSpec((N, 1), lambda b: (0, 0)),
            pl.BlockSpec((N, 1), lambda b: (0, 0)),
            pl.BlockSpec((H, 2 * N), lambda b: (0, 0)),
        ],
        out_specs=pl.BlockSpec(memory_space=pl.MemorySpace.ANY),
        out_shape=jax.ShapeDtypeStruct((B, H, T), jnp.float32),
        scratch_shapes=[
            pltpu.VMEM((4, H, Tt), jnp.float32),
            pltpu.SemaphoreType.DMA((4,)),
        ],
        compiler_params=pltpu.CompilerParams(
            dimension_semantics=("arbitrary",)),
        interpret=interpret,
    )(x, p_hi, e_hi, ctab)


# speedup vs baseline: 1.1675x; 1.1386x over previous
import functools

import jax
import jax.numpy as jnp
from jax.experimental import pallas as pl
from jax.experimental.pallas import tpu as pltpu


def _body(x_ref, phi_ref, ehi_ref, ctab_ref, out_ref):
    nb = x_ref.shape[0]
    zrow = jnp.zeros((1, x_ref.shape[2]), dtype=jnp.bfloat16)
    for i in range(nb):
        vp = x_ref[i, 0:1, :]
        ve = x_ref[i, 1:2, :]
        g_p = (phi_ref[:, :] >= vp).astype(jnp.bfloat16)
        g_e = (ehi_ref[:, :] >= ve).astype(jnp.bfloat16)
        oh_p = g_p - jnp.concatenate([zrow, g_p[:-1, :]], axis=0)
        oh_e = g_e - jnp.concatenate([zrow, g_e[:-1, :]], axis=0)
        oh = jnp.concatenate([oh_p, oh_e], axis=0)
        out_ref[i] = jnp.dot(ctab_ref[:, :], oh,
                             preferred_element_type=jnp.float32)


@functools.partial(jax.jit, static_argnames=("interpret",))
def kernel(x, pitch_bins, energy_bins, pitch_embedding, energy_embedding,
           interpret=False):
    B, _, T = x.shape
    N, H = pitch_embedding.shape
    Tt = 2048
    Bb = 8

    inf = jnp.array([jnp.inf], dtype=jnp.float32)
    p_hi = jnp.concatenate([pitch_bins, inf])[:, None]
    e_hi = jnp.concatenate([energy_bins, inf])[:, None]
    ctab = jnp.concatenate(
        [pitch_embedding.T, energy_embedding.T], axis=1,
    ).astype(jnp.bfloat16)

    grid = (B // Bb, T // Tt)
    return pl.pallas_call(
        _body,
        grid=grid,
        in_specs=[
            pl.BlockSpec((Bb, 2, Tt), lambda b, j: (b, 0, j)),
            pl.BlockSpec((N, 1), lambda b, j: (0, 0)),
            pl.BlockSpec((N, 1), lambda b, j: (0, 0)),
            pl.BlockSpec((H, 2 * N), lambda b, j: (0, 0)),
        ],
        out_specs=pl.BlockSpec((Bb, H, Tt), lambda b, j: (b, 0, j)),
        out_shape=jax.ShapeDtypeStruct((B, H, T), jnp.float32),
        compiler_params=pltpu.CompilerParams(
            dimension_semantics=("parallel", "parallel")),
        interpret=interpret,
    )(x, p_hi, e_hi, ctab)
